# baseline (speedup 1.0000x reference)
import functools

import jax
import jax.numpy as jnp
from jax import lax
from jax.experimental import pallas as pl
from jax.experimental.pallas import tpu as pltpu
from jax.experimental.pallas import tpu_sc as plsc

_B = 1024
_N = 100000
_SPLIT = 512
_RB = 32

_NALIGN = 99968
_NW = 32
_SLABS_PW = 2
_CW = 4096
_NFULL = 24
_TAIL = _NALIGN - _NFULL * _CW


def _tc_kernel(lab_ref, clab_ref, s_ref, out_ref, acc_ref):
    i = pl.program_id(0)
    nt = pl.num_programs(0)

    @pl.when(i == 0)
    def _init():
        acc_ref[0] = 0.0
        acc_ref[1] = 0.0
        acc_ref[2] = 0.0

    lab = lab_ref[:]
    clab = clab_ref[:]
    s = s_ref[:]
    m = lab == clab
    t2 = jnp.maximum(s, 0.0)
    t1 = jnp.maximum(1.0 - s, 0.0)
    acc_ref[0] += jnp.sum(jnp.where(m, 0.0, t2))
    acc_ref[1] += jnp.sum(jnp.where(m, t1, 0.0))
    acc_ref[2] += jnp.sum(m.astype(jnp.float32))

    @pl.when(i == nt - 1)
    def _fin():
        out_ref[0] = acc_ref[0]
        out_ref[1] = acc_ref[1]
        out_ref[2] = acc_ref[2]


def _tc_sums(fuse_scores, lab2d, clab2d):
    return pl.pallas_call(
        _tc_kernel,
        grid=(_SPLIT // _RB,),
        in_specs=[
            pl.BlockSpec((_RB, 1), lambda i: (i, 0)),
            pl.BlockSpec((1, _N), lambda i: (0, 0)),
            pl.BlockSpec((_RB, _N), lambda i: (i, 0)),
        ],
        out_specs=pl.BlockSpec(memory_space=pltpu.SMEM),
        out_shape=jax.ShapeDtypeStruct((3,), jnp.float32),
        scratch_shapes=[pltpu.SMEM((3,), jnp.float32)],
        compiler_params=pltpu.CompilerParams(
            vmem_limit_bytes=128 * 1024 * 1024,
        ),
    )(lab2d, clab2d, fuse_scores)


def _tc_corner_sums(s_corner, lab2d_sc, clab2d_corner):
    w = _N - _NALIGN
    return pl.pallas_call(
        _tc_kernel,
        grid=(1,),
        in_specs=[
            pl.BlockSpec((_B - _SPLIT, 1), lambda i: (0, 0)),
            pl.BlockSpec((1, w), lambda i: (0, 0)),
            pl.BlockSpec((_B - _SPLIT, w), lambda i: (0, 0)),
        ],
        out_specs=pl.BlockSpec(memory_space=pltpu.SMEM),
        out_shape=jax.ShapeDtypeStruct((3,), jnp.float32),
        scratch_shapes=[pltpu.SMEM((3,), jnp.float32)],
    )(lab2d_sc, clab2d_corner, s_corner)


def _sc_chunk_dmas(s_ref, cl_ref, s_buf, cl_buf, sem, row8, off, cw):
    return [
        pltpu.make_async_copy(
            s_ref.at[pl.ds(row8, 8), pl.ds(off, cw)], s_buf.at[:, pl.ds(0, cw)], sem
        ),
        pltpu.make_async_copy(cl_ref.at[pl.ds(off, cw)], cl_buf.at[pl.ds(0, cw)], sem),
    ]


_UNROLL = 8


def _sc_compute_chunk(s_buf, cl_buf, labs8, accs, cw):
    niter = cw // (16 * _UNROLL)

    def row_body(r):
        lab_b = labs8[r]

        def body(v, carry):
            nm, mt, cf = carry
            base = v * 16 * _UNROLL
            for u in range(_UNROLL):
                s = s_buf[r, pl.ds(base + u * 16, 16)]
                cl = cl_buf[pl.ds(base + u * 16, 16)]
                m = cl == lab_b
                t2 = jnp.maximum(s, 0.0)
                t1 = jnp.maximum(1.0 - s, 0.0)
                nm = nm + jnp.where(m, 0.0, t2)
                mt = mt + jnp.where(m, t1, 0.0)
                cf = cf + jnp.where(m, 1.0, 0.0)
            return nm, mt, cf

        return body

    for r in range(8):
        accs = lax.fori_loop(0, niter, row_body(r), accs)
    return accs


def _sc_worker(s_ref, labels_ref, cl_ref, out_ref,
               lab_v, s_buf0, s_buf1, cl_buf0, cl_buf1, res_v, sem0, sem1):
    cid = lax.axis_index("c")
    sid = lax.axis_index("s")
    wid = (sid * 2 + cid).astype(jnp.int32)

    pltpu.sync_copy(labels_ref.at[pl.ds(wid * 8 * _SLABS_PW, 16)], lab_v)
    lab16 = lab_v[...]

    zero = jnp.zeros((16,), jnp.float32)
    accs = (zero, zero, zero)

    for j in range(_SLABS_PW):
        slab = wid * _SLABS_PW + j
        row8 = _SPLIT + slab * 8
        labs8 = [lab16[j * 8 + r] for r in range(8)]

        def fire(s_b, cl_b, sem, off, cw):
            for c in _sc_chunk_dmas(s_ref, cl_ref, s_b, cl_b, sem, row8, off, cw):
                c.start()

        def drain(s_b, cl_b, sem, off, cw):
            for c in _sc_chunk_dmas(s_ref, cl_ref, s_b, cl_b, sem, row8, off, cw):
                c.wait()

        fire(s_buf0, cl_buf0, sem0, 0, _CW)

        def round_body(k, accs):
            off = k * 2 * _CW
            fire(s_buf1, cl_buf1, sem1, off + _CW, _CW)
            drain(s_buf0, cl_buf0, sem0, off, _CW)
            accs = _sc_compute_chunk(s_buf0, cl_buf0, labs8, accs, _CW)

            @pl.when(k < 11)
            def _():
                fire(s_buf0, cl_buf0, sem0, off + 2 * _CW, _CW)

            drain(s_buf1, cl_buf1, sem1, off + _CW, _CW)
            accs = _sc_compute_chunk(s_buf1, cl_buf1, labs8, accs, _CW)
            return accs

        accs = lax.fori_loop(0, 12, round_body, accs)

        fire(s_buf0, cl_buf0, sem0, _NFULL * _CW, _TAIL)
        drain(s_buf0, cl_buf0, sem0, _NFULL * _CW, _TAIL)
        accs = _sc_compute_chunk(s_buf0, cl_buf0, labs8, accs, _TAIL)

    res_v[pl.ds(0, 16)] = accs[0]
    res_v[pl.ds(16, 16)] = accs[1]
    res_v[pl.ds(32, 16)] = accs[2]
    pltpu.sync_copy(res_v, out_ref.at[pl.ds(wid * 48, 48)])


def _sc_sums(fuse_scores, labels_sc, clab):
    kfn = pl.kernel(
        _sc_worker,
        out_type=jax.ShapeDtypeStruct((_NW * 48,), jnp.float32),
        mesh=plsc.VectorSubcoreMesh(core_axis_name="c", subcore_axis_name="s"),
        scratch_types=[
            pltpu.VMEM((16,), jnp.int32),
            pltpu.VMEM((8, _CW), jnp.float32),
            pltpu.VMEM((8, _CW), jnp.float32),
            pltpu.VMEM((_CW,), jnp.int32),
            pltpu.VMEM((_CW,), jnp.int32),
            pltpu.VMEM((48,), jnp.float32),
            pltpu.SemaphoreType.DMA,
            pltpu.SemaphoreType.DMA,
        ],
        compiler_params=pltpu.CompilerParams(use_tc_tiling_on_sc=True),
    )
    return kfn(fuse_scores, labels_sc, clab)


def kernel(fuse_scores, labels, center_labels):
    with jax.enable_x64(False):
        return _run(fuse_scores, labels, center_labels)


def _run(fuse_scores, labels, center_labels):
    B, N = fuse_scores.shape
    lab32 = labels.astype(jnp.int32)
    clab32 = center_labels.astype(jnp.int32)
    lab2d = lab32.reshape(B, 1)
    clab2d = clab32.reshape(1, N)

    lab_sc = jnp.pad(lab32[_SPLIT:], (0, 16))
    sc = _sc_sums(fuse_scores, lab_sc, clab32).reshape(_NW, 3, 16)
    corner = _tc_corner_sums(
        fuse_scores[_SPLIT:, _NALIGN:],
        lab2d[_SPLIT:],
        clab2d[:, _NALIGN:],
    )
    tc = _tc_sums(fuse_scores, lab2d, clab2d)

    nm = tc[0] + corner[0] + jnp.sum(sc[:, 0, :])
    mt = tc[1] + corner[1] + jnp.sum(sc[:, 1, :])
    cnt = tc[2] + corner[2] + jnp.sum(sc[:, 2, :])
    n_non = jnp.float32(float(B) * float(N)) - cnt
    score = mt / cnt + nm / n_non
    return (score, score)


# speedup vs baseline: 1.0502x; 1.0502x over previous
import functools

import jax
import jax.numpy as jnp
from jax import lax
from jax.experimental import pallas as pl
from jax.experimental.pallas import tpu as pltpu
from jax.experimental.pallas import tpu_sc as plsc

_B = 1024
_N = 100000
_SPLIT = 768
_RB = 48

_NALIGN = 99968
_NW = 32
_SLABS_PW = 1
_CW = 4096
_NFULL = 24
_TAIL = _NALIGN - _NFULL * _CW


def _tc_kernel(lab_ref, clab_ref, s_ref, out_ref, acc_ref):
    i = pl.program_id(0)
    nt = pl.num_programs(0)

    @pl.when(i == 0)
    def _init():
        acc_ref[0] = 0.0
        acc_ref[1] = 0.0
        acc_ref[2] = 0.0

    lab = lab_ref[:]
    clab = clab_ref[:]
    s = s_ref[:]
    m = lab == clab
    t2 = jnp.maximum(s, 0.0)
    t1 = jnp.maximum(1.0 - s, 0.0)
    acc_ref[0] += jnp.sum(jnp.where(m, 0.0, t2))
    acc_ref[1] += jnp.sum(jnp.where(m, t1, 0.0))
    acc_ref[2] += jnp.sum(m.astype(jnp.float32))

    @pl.when(i == nt - 1)
    def _fin():
        out_ref[0] = acc_ref[0]
        out_ref[1] = acc_ref[1]
        out_ref[2] = acc_ref[2]


def _tc_sums(fuse_scores, lab2d, clab2d):
    return pl.pallas_call(
        _tc_kernel,
        grid=(_SPLIT // _RB,),
        in_specs=[
            pl.BlockSpec((_RB, 1), lambda i: (i, 0)),
            pl.BlockSpec((1, _N), lambda i: (0, 0)),
            pl.BlockSpec((_RB, _N), lambda i: (i, 0)),
        ],
        out_specs=pl.BlockSpec(memory_space=pltpu.SMEM),
        out_shape=jax.ShapeDtypeStruct((3,), jnp.float32),
        scratch_shapes=[pltpu.SMEM((3,), jnp.float32)],
        compiler_params=pltpu.CompilerParams(
            vmem_limit_bytes=128 * 1024 * 1024,
        ),
    )(lab2d, clab2d, fuse_scores)


def _tc_corner_sums(s_corner, lab2d_sc, clab2d_corner):
    w = _N - _NALIGN
    return pl.pallas_call(
        _tc_kernel,
        grid=(1,),
        in_specs=[
            pl.BlockSpec((_B - _SPLIT, 1), lambda i: (0, 0)),
            pl.BlockSpec((1, w), lambda i: (0, 0)),
            pl.BlockSpec((_B - _SPLIT, w), lambda i: (0, 0)),
        ],
        out_specs=pl.BlockSpec(memory_space=pltpu.SMEM),
        out_shape=jax.ShapeDtypeStruct((3,), jnp.float32),
        scratch_shapes=[pltpu.SMEM((3,), jnp.float32)],
    )(lab2d_sc, clab2d_corner, s_corner)


def _sc_chunk_dmas(s_ref, cl_ref, s_buf, cl_buf, sem, row8, off, cw):
    return [
        pltpu.make_async_copy(
            s_ref.at[pl.ds(row8, 8), pl.ds(off, cw)], s_buf.at[:, pl.ds(0, cw)], sem
        ),
        pltpu.make_async_copy(cl_ref.at[pl.ds(off, cw)], cl_buf.at[pl.ds(0, cw)], sem),
    ]


_UNROLL = 8


def _sc_compute_chunk(s_buf, cl_buf, labs8, accs, cw):
    niter = cw // (16 * _UNROLL)

    def row_body(r):
        lab_b = labs8[r]

        def body(v, carry):
            nm, mt, cf = carry
            base = v * 16 * _UNROLL
            for u in range(_UNROLL):
                s = s_buf[r, pl.ds(base + u * 16, 16)]
                cl = cl_buf[pl.ds(base + u * 16, 16)]
                m = cl == lab_b
                t2 = jnp.maximum(s, 0.0)
                t1 = jnp.maximum(1.0 - s, 0.0)
                nm = nm + jnp.where(m, 0.0, t2)
                mt = mt + jnp.where(m, t1, 0.0)
                cf = cf + jnp.where(m, 1.0, 0.0)
            return nm, mt, cf

        return body

    for r in range(8):
        accs = lax.fori_loop(0, niter, row_body(r), accs)
    return accs


def _sc_worker(s_ref, labels_ref, cl_ref, out_ref,
               lab_v, s_buf0, s_buf1, cl_buf0, cl_buf1, res_v, sem0, sem1):
    cid = lax.axis_index("c")
    sid = lax.axis_index("s")
    wid = (sid * 2 + cid).astype(jnp.int32)

    pltpu.sync_copy(labels_ref.at[pl.ds(wid * 8 * _SLABS_PW, 16)], lab_v)
    lab16 = lab_v[...]

    zero = jnp.zeros((16,), jnp.float32)
    accs = (zero, zero, zero)

    for j in range(_SLABS_PW):
        slab = wid * _SLABS_PW + j
        row8 = _SPLIT + slab * 8
        labs8 = [lab16[j * 8 + r] for r in range(8)]

        def fire(s_b, cl_b, sem, off, cw):
            for c in _sc_chunk_dmas(s_ref, cl_ref, s_b, cl_b, sem, row8, off, cw):
                c.start()

        def drain(s_b, cl_b, sem, off, cw):
            for c in _sc_chunk_dmas(s_ref, cl_ref, s_b, cl_b, sem, row8, off, cw):
                c.wait()

        fire(s_buf0, cl_buf0, sem0, 0, _CW)

        def round_body(k, accs):
            off = k * 2 * _CW
            fire(s_buf1, cl_buf1, sem1, off + _CW, _CW)
            drain(s_buf0, cl_buf0, sem0, off, _CW)
            accs = _sc_compute_chunk(s_buf0, cl_buf0, labs8, accs, _CW)

            @pl.when(k < 11)
            def _():
                fire(s_buf0, cl_buf0, sem0, off + 2 * _CW, _CW)

            drain(s_buf1, cl_buf1, sem1, off + _CW, _CW)
            accs = _sc_compute_chunk(s_buf1, cl_buf1, labs8, accs, _CW)
            return accs

        accs = lax.fori_loop(0, 12, round_body, accs)

        fire(s_buf0, cl_buf0, sem0, _NFULL * _CW, _TAIL)
        drain(s_buf0, cl_buf0, sem0, _NFULL * _CW, _TAIL)
        accs = _sc_compute_chunk(s_buf0, cl_buf0, labs8, accs, _TAIL)

    res_v[pl.ds(0, 16)] = accs[0]
    res_v[pl.ds(16, 16)] = accs[1]
    res_v[pl.ds(32, 16)] = accs[2]
    pltpu.sync_copy(res_v, out_ref.at[pl.ds(wid * 48, 48)])


def _sc_sums(fuse_scores, labels_sc, clab):
    kfn = pl.kernel(
        _sc_worker,
        out_type=jax.ShapeDtypeStruct((_NW * 48,), jnp.float32),
        mesh=plsc.VectorSubcoreMesh(core_axis_name="c", subcore_axis_name="s"),
        scratch_types=[
            pltpu.VMEM((16,), jnp.int32),
            pltpu.VMEM((8, _CW), jnp.float32),
            pltpu.VMEM((8, _CW), jnp.float32),
            pltpu.VMEM((_CW,), jnp.int32),
            pltpu.VMEM((_CW,), jnp.int32),
            pltpu.VMEM((48,), jnp.float32),
            pltpu.SemaphoreType.DMA,
            pltpu.SemaphoreType.DMA,
        ],
        compiler_params=pltpu.CompilerParams(use_tc_tiling_on_sc=True),
    )
    return kfn(fuse_scores, labels_sc, clab)


def kernel(fuse_scores, labels, center_labels):
    with jax.enable_x64(False):
        return _run(fuse_scores, labels, center_labels)


def _run(fuse_scores, labels, center_labels):
    B, N = fuse_scores.shape
    lab32 = labels.astype(jnp.int32)
    clab32 = center_labels.astype(jnp.int32)
    lab2d = lab32.reshape(B, 1)
    clab2d = clab32.reshape(1, N)

    lab_sc = jnp.pad(lab32[_SPLIT:], (0, 16))
    sc = _sc_sums(fuse_scores, lab_sc, clab32).reshape(_NW, 3, 16)
    corner = _tc_corner_sums(
        fuse_scores[_SPLIT:, _NALIGN:],
        lab2d[_SPLIT:],
        clab2d[:, _NALIGN:],
    )
    tc = _tc_sums(fuse_scores, lab2d, clab2d)

    nm = tc[0] + corner[0] + jnp.sum(sc[:, 0, :])
    mt = tc[1] + corner[1] + jnp.sum(sc[:, 1, :])
    cnt = tc[2] + corner[2] + jnp.sum(sc[:, 2, :])
    n_non = jnp.float32(float(B) * float(N)) - cnt
    score = mt / cnt + nm / n_non
    return (score, score)
